# baseline (speedup 1.0000x reference)
import functools

import jax
import jax.numpy as jnp
from jax import lax
from jax.experimental import pallas as pl
from jax.experimental.pallas import tpu as pltpu
from jax.experimental.pallas import tpu_sc as plsc

N, D = 262144, 256
H = D // 2
N2 = 2 * N
NUM_CORES = 2
NUM_SUBCORES = 16
NW = NUM_CORES * NUM_SUBCORES
ROWS_PER_W = N // NW
R = 128
CHUNKS = ROWS_PER_W // R
NBUF = 3

_mesh = plsc.VectorSubcoreMesh(core_axis_name="c", subcore_axis_name="s")


@functools.partial(
    pl.kernel,
    mesh=_mesh,
    out_type=jax.ShapeDtypeStruct((N, H), jnp.float32),
    scratch_types=[
        pltpu.VMEM((NBUF, R, H), jnp.float32),
        pltpu.VMEM((NBUF, R), jnp.int32),
        pltpu.SemaphoreType.DMA,
        pltpu.SemaphoreType.DMA,
    ],
)
def _sc_speed(inp2_hbm, speed_hbm, buf, idx, in_sem, out_sem):
    wid = lax.axis_index("s") * NUM_CORES + lax.axis_index("c")
    base = wid * ROWS_PER_W
    lanes = lax.iota(jnp.int32, 16)

    def rows(i):
        return pl.ds(base + i * R, R)

    def fill_idx(i, slot):
        start = 2 * (base + i * R)
        for k in range(R // 16):
            idx[slot, pl.ds(k * 16, 16)] = start + 2 * (k * 16 + lanes)

    def start_gather(i, slot):
        fill_idx(i, slot)
        pltpu.async_copy(inp2_hbm.at[idx.at[slot]], buf.at[slot], in_sem)

    def wait_gather(i, slot):
        pltpu.make_async_copy(inp2_hbm.at[idx.at[slot]], buf.at[slot], in_sem).wait()

    def start_write(i, slot):
        pltpu.async_copy(buf.at[slot], speed_hbm.at[rows(i)], out_sem)

    def wait_write(i, slot):
        pltpu.make_async_copy(buf.at[slot], speed_hbm.at[rows(i)], out_sem).wait()

    for j in range(NBUF):
        start_gather(j, j)

    def body(i, _):
        slot = lax.rem(i, NBUF)

        @pl.when(i >= 1)
        def _():
            prev_slot = lax.rem(i - 1, NBUF)
            wait_write(i - 1, prev_slot)

            @pl.when(i - 1 + NBUF < CHUNKS)
            def _():
                start_gather(i - 1 + NBUF, prev_slot)

        wait_gather(i, slot)
        start_write(i, slot)
        return 0

    lax.fori_loop(0, CHUNKS, body, 0)
    wait_write(CHUNKS - 1, lax.rem(CHUNKS - 1, NBUF))


BR = 2048


def _tc_copy_body(x_ref, o_ref):
    o_ref[...] = x_ref[...]


_tc_dir = pl.pallas_call(
    _tc_copy_body,
    grid=(N // BR,),
    in_specs=[pl.BlockSpec((BR, H), lambda i: (i, 1))],
    out_specs=pl.BlockSpec((BR, H), lambda i: (i, 0)),
    out_shape=jax.ShapeDtypeStruct((N, H), jnp.float32),
)


def kernel(inputs):
    speed = _sc_speed(inputs.reshape(N2, H))
    direction = _tc_dir(inputs)
    return (speed, direction)


# speedup vs baseline: 2.2020x; 2.2020x over previous
import functools

import jax
import jax.numpy as jnp
from jax import lax
from jax.experimental import pallas as pl
from jax.experimental.pallas import tpu as pltpu
from jax.experimental.pallas import tpu_sc as plsc

N, D = 262144, 256
H = D // 2
NUM_CORES = 2
NUM_SUBCORES = 16
NW = NUM_CORES * NUM_SUBCORES
ROWS_PER_W = N // NW
R = 128
CHUNKS = ROWS_PER_W // R
NBUF = 3

_mesh = plsc.VectorSubcoreMesh(core_axis_name="c", subcore_axis_name="s")


@functools.partial(
    pl.kernel,
    mesh=_mesh,
    out_type=jax.ShapeDtypeStruct((N, H), jnp.float32),
    scratch_types=[
        pltpu.VMEM((NBUF, R, H), jnp.float32),
        pltpu.SemaphoreType.DMA,
        pltpu.SemaphoreType.DMA,
    ],
)
def _sc_speed(inp_hbm, speed_hbm, buf, in_sem, out_sem):
    wid = lax.axis_index("s") * NUM_CORES + lax.axis_index("c")
    base = wid * ROWS_PER_W

    def rows(i):
        return pl.ds(base + i * R, R)

    def start_read(i, slot):
        pltpu.async_copy(inp_hbm.at[rows(i), pl.ds(0, H)], buf.at[slot], in_sem)

    def wait_read(i, slot):
        pltpu.make_async_copy(inp_hbm.at[rows(i), pl.ds(0, H)], buf.at[slot], in_sem).wait()

    def start_write(i, slot):
        pltpu.async_copy(buf.at[slot], speed_hbm.at[rows(i)], out_sem)

    def wait_write(i, slot):
        pltpu.make_async_copy(buf.at[slot], speed_hbm.at[rows(i)], out_sem).wait()

    for j in range(NBUF):
        start_read(j, j)

    def body(i, _):
        slot = lax.rem(i, NBUF)

        @pl.when(i >= 1)
        def _():
            prev_slot = lax.rem(i - 1, NBUF)
            wait_write(i - 1, prev_slot)

            @pl.when(i - 1 + NBUF < CHUNKS)
            def _():
                start_read(i - 1 + NBUF, prev_slot)

        wait_read(i, slot)
        start_write(i, slot)
        return 0

    lax.fori_loop(0, CHUNKS, body, 0)
    wait_write(CHUNKS - 1, lax.rem(CHUNKS - 1, NBUF))


BR = 2048


def _tc_copy_body(x_ref, o_ref):
    o_ref[...] = x_ref[...]


_tc_dir = pl.pallas_call(
    _tc_copy_body,
    grid=(N // BR,),
    in_specs=[pl.BlockSpec((BR, H), lambda i: (i, 1))],
    out_specs=pl.BlockSpec((BR, H), lambda i: (i, 0)),
    out_shape=jax.ShapeDtypeStruct((N, H), jnp.float32),
)


def kernel(inputs):
    speed = _sc_speed(inputs)
    direction = _tc_dir(inputs)
    return (speed, direction)


# speedup vs baseline: 2.3919x; 1.0862x over previous
import functools

import jax
import jax.numpy as jnp
from jax import lax
from jax.experimental import pallas as pl
from jax.experimental.pallas import tpu as pltpu
from jax.experimental.pallas import tpu_sc as plsc

N, D = 262144, 256
H = D // 2
NUM_CORES = 2
NUM_SUBCORES = 16
NW = NUM_CORES * NUM_SUBCORES
ROWS_PER_W = N // NW
R = 128
CHUNKS = ROWS_PER_W // R
NBUF = 3

_mesh = plsc.VectorSubcoreMesh(core_axis_name="c", subcore_axis_name="s")


@functools.partial(
    pl.kernel,
    mesh=_mesh,
    out_type=(
        jax.ShapeDtypeStruct((N, H), jnp.float32),
        jax.ShapeDtypeStruct((N, H), jnp.float32),
    ),
    scratch_types=[
        pltpu.MemorySpace.VMEM_SHARED((NUM_SUBCORES, NBUF, R, D), jnp.float32),
        pltpu.SemaphoreType.DMA,
        pltpu.SemaphoreType.DMA,
    ],
)
def _split_halves(inp_hbm, speed_hbm, dir_hbm, shared, in_sem, out_sem):
    cid = lax.axis_index("c")
    sid = lax.axis_index("s")
    wid = sid * NUM_CORES + cid
    base = wid * ROWS_PER_W

    def rows(i):
        return pl.ds(base + i * R, R)

    def start_read(i, slot):
        pltpu.async_copy(inp_hbm.at[rows(i)], shared.at[sid, slot], in_sem)

    def wait_read(i, slot):
        pltpu.make_async_copy(inp_hbm.at[rows(i)], shared.at[sid, slot], in_sem).wait()

    def start_writes(i, slot):
        pltpu.async_copy(shared.at[sid, slot, :, pl.ds(0, H)], speed_hbm.at[rows(i)], out_sem)
        pltpu.async_copy(shared.at[sid, slot, :, pl.ds(H, H)], dir_hbm.at[rows(i)], out_sem)

    def wait_writes(i, slot):
        pltpu.make_async_copy(shared.at[sid, slot, :, pl.ds(0, H)], speed_hbm.at[rows(i)], out_sem).wait()
        pltpu.make_async_copy(shared.at[sid, slot, :, pl.ds(H, H)], dir_hbm.at[rows(i)], out_sem).wait()

    for j in range(NBUF):
        start_read(j, j)

    def body(i, _):
        slot = lax.rem(i, NBUF)

        @pl.when(i >= 1)
        def _():
            prev_slot = lax.rem(i - 1, NBUF)
            wait_writes(i - 1, prev_slot)

            @pl.when(i - 1 + NBUF < CHUNKS)
            def _():
                start_read(i - 1 + NBUF, prev_slot)

        wait_read(i, slot)
        start_writes(i, slot)
        return 0

    lax.fori_loop(0, CHUNKS, body, 0)
    wait_writes(CHUNKS - 1, lax.rem(CHUNKS - 1, NBUF))


def kernel(inputs):
    return _split_halves(inputs)
